# baseline (speedup 1.0000x reference)
import functools

import jax
import jax.numpy as jnp
from jax import lax
from jax.experimental import pallas as pl
from jax.experimental.pallas import tpu as pltpu
from jax.experimental.pallas import tpu_sc as plsc

VOCAB = 1000000
DIM = 64
B = 16384
L = 50

NC = 2
NS = 16
NW = NC * NS
SEQ_PER_W = B // NW
GROUP_SEQS = 4
GROUP_IDX = GROUP_SEQS * L
GROUPS_PER_W = SEQ_PER_W // GROUP_SEQS
NBUF = 3
IDX_PER_W = SEQ_PER_W * L


def _pool_body(table_hbm, xf_hbm, out_hbm, idx_v, b0, b1, b2, o0, o1, o2,
               s0, s1, s2, t0, t1, t2):
    bufs = (b0, b1, b2)
    sems = (s0, s1, s2)
    obufs = (o0, o1, o2)
    osems = (t0, t1, t2)
    wid = lax.axis_index("s") * NC + lax.axis_index("c")
    idx_base = wid * IDX_PER_W
    out_base = wid * SEQ_PER_W

    pltpu.sync_copy(xf_hbm.at[pl.ds(wid * SEQ_PER_W, SEQ_PER_W)],
                    idx_v)

    def fire(j, b):
        for i in range(GROUP_SEQS):
            pltpu.async_copy(
                table_hbm.at[idx_v.at[j * GROUP_SEQS + i]],
                bufs[b].at[i], sems[b])

    for b in range(NBUF):
        fire(b, b)

    zero = jnp.zeros((16,), jnp.float32)
    scale = jnp.float32(1.0 / L)

    def drain(b):
        for i in range(GROUP_SEQS):
            pltpu.make_async_copy(
                table_hbm.at[idx_v.at[0]], bufs[b].at[i],
                sems[b]).wait()

    def odrain(ob):
        pltpu.make_async_copy(
            obufs[ob], out_hbm.at[pl.ds(out_base, GROUP_SEQS)],
            osems[ob]).wait()

    def accumulate(j, b, ob):
        buf = bufs[b]
        obuf = obufs[ob]
        for s in range(GROUP_SEQS):
            def rbody(r, accs):
                a0, a1, a2, a3 = accs
                row = r * 2
                a0 = a0 + buf[s, row, pl.ds(0, 16)]
                a1 = a1 + buf[s, row, pl.ds(16, 16)]
                a2 = a2 + buf[s, row, pl.ds(32, 16)]
                a3 = a3 + buf[s, row, pl.ds(48, 16)]
                a0 = a0 + buf[s, row + 1, pl.ds(0, 16)]
                a1 = a1 + buf[s, row + 1, pl.ds(16, 16)]
                a2 = a2 + buf[s, row + 1, pl.ds(32, 16)]
                a3 = a3 + buf[s, row + 1, pl.ds(48, 16)]
                return (a0, a1, a2, a3)

            a0, a1, a2, a3 = lax.fori_loop(0, L // 2, rbody,
                                           (zero, zero, zero, zero))
            obuf[s, pl.ds(0, 16)] = a0 * scale
            obuf[s, pl.ds(16, 16)] = a1 * scale
            obuf[s, pl.ds(32, 16)] = a2 * scale
            obuf[s, pl.ds(48, 16)] = a3 * scale
        pltpu.async_copy(
            obuf, out_hbm.at[pl.ds(out_base + j * GROUP_SEQS, GROUP_SEQS)],
            osems[ob])

    def outer(jo, carry):
        for b in range(NBUF):
            j = jo * NBUF + b
            drain(b)

            @pl.when(j >= NBUF)
            def _():
                odrain(b)

            accumulate(j, b, b)

            @pl.when(j + NBUF < GROUPS_PER_W)
            def _():
                fire(j + NBUF, b)
        return carry

    lax.fori_loop(0, GROUPS_PER_W // NBUF, outer, 0)

    for j in range((GROUPS_PER_W // NBUF) * NBUF, GROUPS_PER_W):
        b = j % NBUF
        drain(b)
        odrain(b)
        accumulate(j, b, b)
    for b in range(NBUF):
        odrain(b)


def _pooled(table, xf):
    mesh = plsc.VectorSubcoreMesh(core_axis_name="c", subcore_axis_name="s",
                                  num_cores=NC, num_subcores=NS)
    return pl.kernel(
        _pool_body,
        out_type=jax.ShapeDtypeStruct((B, DIM), jnp.float32),
        mesh=mesh,
        compiler_params=pltpu.CompilerParams(use_tc_tiling_on_sc=False),
        scratch_types=[
            pltpu.VMEM((SEQ_PER_W, L), jnp.int32),
            pltpu.VMEM((GROUP_SEQS, L, 2 * DIM), jnp.float32),
            pltpu.VMEM((GROUP_SEQS, L, 2 * DIM), jnp.float32),
            pltpu.VMEM((GROUP_SEQS, L, 2 * DIM), jnp.float32),
            pltpu.VMEM((GROUP_SEQS, DIM), jnp.float32),
            pltpu.VMEM((GROUP_SEQS, DIM), jnp.float32),
            pltpu.VMEM((GROUP_SEQS, DIM), jnp.float32),
            pltpu.SemaphoreType.DMA,
            pltpu.SemaphoreType.DMA,
            pltpu.SemaphoreType.DMA,
            pltpu.SemaphoreType.DMA,
            pltpu.SemaphoreType.DMA,
            pltpu.SemaphoreType.DMA,
        ],
    )(table, xf)


def _heads_body(p_ref, wc_ref, bc_ref, w1_ref, b1_ref, w2_ref, b2_ref,
                cor_ref, inc_ref):
    p = p_ref[:, :]
    cn = (((1,), (1,)), ((), ()))
    cor_ref[:, :] = (lax.dot_general(p, wc_ref[:, :], cn,
                                     preferred_element_type=jnp.float32)
                     + bc_ref[:, :])
    h = (lax.dot_general(p, w1_ref[:, :], cn,
                         preferred_element_type=jnp.float32)
         + b1_ref[:, :])
    h = 0.5 * h * (1.0 + lax.erf(h * (2.0 ** -0.5)))
    inc_ref[:, :] = (lax.dot_general(h, w2_ref[:, :], cn,
                                     preferred_element_type=jnp.float32)
                     + b2_ref[:, :])


def _heads(pooled, Wc, bc, W1, b1, W2, b2):
    BS = 2048
    grid = (B // BS,)
    wspec = pl.BlockSpec((DIM, DIM), lambda i: (0, 0))
    bspec = pl.BlockSpec((1, DIM), lambda i: (0, 0))
    pspec = pl.BlockSpec((BS, DIM), lambda i: (i, 0))
    return pl.pallas_call(
        _heads_body,
        grid=grid,
        in_specs=[pspec, wspec, bspec, wspec, bspec, wspec, bspec],
        out_specs=[pspec, pspec],
        out_shape=[jax.ShapeDtypeStruct((B, DIM), jnp.float32),
                   jax.ShapeDtypeStruct((B, DIM), jnp.float32)],
    )(pooled, Wc, bc.reshape(1, DIM), W1, b1.reshape(1, DIM), W2,
      b2.reshape(1, DIM))


def kernel(x, table, Wc, bc, W1, b1, W2, b2):
    tpad = jnp.pad(table, ((0, 0), (0, DIM)))
    pooled = _pooled(tpad, x)
    cor, inc = _heads(pooled, Wc, bc, W1, b1, W2, b2)
    return (cor, inc)


# speedup vs baseline: 1.0754x; 1.0754x over previous
import functools

import jax
import jax.numpy as jnp
from jax import lax
from jax.experimental import pallas as pl
from jax.experimental.pallas import tpu as pltpu
from jax.experimental.pallas import tpu_sc as plsc

VOCAB = 1000000
DIM = 64
B = 16384
L = 50

NC = 2
NS = 16
NW = NC * NS
SEQ_PER_W = B // NW
GROUP_SEQS = 8
GROUP_IDX = GROUP_SEQS * L
GROUPS_PER_W = SEQ_PER_W // GROUP_SEQS
NBUF = 3
IDX_PER_W = SEQ_PER_W * L


def _pool_body(table_hbm, xf_hbm, out_hbm, idx_v, b0, b1, b2, o0, o1, o2,
               s0, s1, s2, t0, t1, t2):
    bufs = (b0, b1, b2)
    sems = (s0, s1, s2)
    obufs = (o0, o1, o2)
    osems = (t0, t1, t2)
    wid = lax.axis_index("s") * NC + lax.axis_index("c")
    idx_base = wid * IDX_PER_W
    out_base = wid * SEQ_PER_W

    pltpu.sync_copy(xf_hbm.at[pl.ds(wid * SEQ_PER_W, SEQ_PER_W)],
                    idx_v)

    def fire(j, b):
        for i in range(GROUP_SEQS):
            pltpu.async_copy(
                table_hbm.at[idx_v.at[j * GROUP_SEQS + i]],
                bufs[b].at[i], sems[b])

    for b in range(NBUF):
        fire(b, b)

    zero = jnp.zeros((16,), jnp.float32)
    scale = jnp.float32(1.0 / L)

    def drain(b):
        for i in range(GROUP_SEQS):
            pltpu.make_async_copy(
                table_hbm.at[idx_v.at[0]], bufs[b].at[i],
                sems[b]).wait()

    def odrain(ob):
        pltpu.make_async_copy(
            obufs[ob], out_hbm.at[pl.ds(out_base, GROUP_SEQS)],
            osems[ob]).wait()

    def accumulate(j, b, ob):
        buf = bufs[b]
        obuf = obufs[ob]
        for s in range(GROUP_SEQS):
            def rbody(r, accs):
                a0, a1, a2, a3 = accs
                row = r * 2
                a0 = a0 + buf[s, row, pl.ds(0, 16)]
                a1 = a1 + buf[s, row, pl.ds(16, 16)]
                a2 = a2 + buf[s, row, pl.ds(32, 16)]
                a3 = a3 + buf[s, row, pl.ds(48, 16)]
                a0 = a0 + buf[s, row + 1, pl.ds(0, 16)]
                a1 = a1 + buf[s, row + 1, pl.ds(16, 16)]
                a2 = a2 + buf[s, row + 1, pl.ds(32, 16)]
                a3 = a3 + buf[s, row + 1, pl.ds(48, 16)]
                return (a0, a1, a2, a3)

            a0, a1, a2, a3 = lax.fori_loop(0, L // 2, rbody,
                                           (zero, zero, zero, zero))
            obuf[s, pl.ds(0, 16)] = a0 * scale
            obuf[s, pl.ds(16, 16)] = a1 * scale
            obuf[s, pl.ds(32, 16)] = a2 * scale
            obuf[s, pl.ds(48, 16)] = a3 * scale
        pltpu.async_copy(
            obuf, out_hbm.at[pl.ds(out_base + j * GROUP_SEQS, GROUP_SEQS)],
            osems[ob])

    def outer(jo, carry):
        for b in range(NBUF):
            j = jo * NBUF + b
            drain(b)

            @pl.when(j >= NBUF)
            def _():
                odrain(b)

            accumulate(j, b, b)

            @pl.when(j + NBUF < GROUPS_PER_W)
            def _():
                fire(j + NBUF, b)
        return carry

    lax.fori_loop(0, GROUPS_PER_W // NBUF, outer, 0)

    for j in range((GROUPS_PER_W // NBUF) * NBUF, GROUPS_PER_W):
        b = j % NBUF
        drain(b)
        odrain(b)
        accumulate(j, b, b)
    for b in range(NBUF):
        odrain(b)


def _pooled(table, xf):
    mesh = plsc.VectorSubcoreMesh(core_axis_name="c", subcore_axis_name="s",
                                  num_cores=NC, num_subcores=NS)
    return pl.kernel(
        _pool_body,
        out_type=jax.ShapeDtypeStruct((B, 2 * DIM), jnp.float32),
        mesh=mesh,
        compiler_params=pltpu.CompilerParams(use_tc_tiling_on_sc=False),
        scratch_types=[
            pltpu.VMEM((SEQ_PER_W, L), jnp.int32),
            pltpu.VMEM((GROUP_SEQS, L, DIM), jnp.float32),
            pltpu.VMEM((GROUP_SEQS, L, DIM), jnp.float32),
            pltpu.VMEM((GROUP_SEQS, L, DIM), jnp.float32),
            pltpu.VMEM((GROUP_SEQS, 2 * DIM), jnp.float32),
            pltpu.VMEM((GROUP_SEQS, 2 * DIM), jnp.float32),
            pltpu.VMEM((GROUP_SEQS, 2 * DIM), jnp.float32),
            pltpu.SemaphoreType.DMA,
            pltpu.SemaphoreType.DMA,
            pltpu.SemaphoreType.DMA,
            pltpu.SemaphoreType.DMA,
            pltpu.SemaphoreType.DMA,
            pltpu.SemaphoreType.DMA,
        ],
    )(table, xf)


def _heads_body(p_ref, wc_ref, bc_ref, w1_ref, b1_ref, w2_ref, b2_ref,
                cor_ref, inc_ref):
    pt = p_ref[:, 0:DIM]
    cnT = (((1,), (1,)), ((), ()))
    cnM = (((1,), (0,)), ((), ()))
    cor_ref[:, :] = (lax.dot_general(wc_ref[:, :], pt, cnT,
                                     preferred_element_type=jnp.float32)
                     + bc_ref[:, :])
    h = (lax.dot_general(w1_ref[:, :], pt, cnT,
                         preferred_element_type=jnp.float32)
         + b1_ref[:, :])
    h = 0.5 * h * (1.0 + lax.erf(h * (2.0 ** -0.5)))
    inc_ref[:, :] = (lax.dot_general(w2_ref[:, :], h, cnM,
                                     preferred_element_type=jnp.float32)
                     + b2_ref[:, :])


def _heads(pooled, Wc, bc, W1, b1, W2, b2):
    BS = 2048
    grid = (B // BS,)
    wspec = pl.BlockSpec((DIM, DIM), lambda i: (0, 0))
    bspec = pl.BlockSpec((DIM, 1), lambda i: (0, 0))
    pspec = pl.BlockSpec((BS, 2 * DIM), lambda i: (i, 0))
    ospec = pl.BlockSpec((DIM, BS), lambda i: (0, i))
    cor_t, inc_t = pl.pallas_call(
        _heads_body,
        grid=grid,
        in_specs=[pspec, wspec, bspec, wspec, bspec, wspec, bspec],
        out_specs=[ospec, ospec],
        out_shape=[jax.ShapeDtypeStruct((DIM, B), jnp.float32),
                   jax.ShapeDtypeStruct((DIM, B), jnp.float32)],
    )(pooled, Wc, bc.reshape(DIM, 1), W1, b1.reshape(DIM, 1), W2,
      b2.reshape(DIM, 1))
    return cor_t.T, inc_t.T


def kernel(x, table, Wc, bc, W1, b1, W2, b2):
    pooled = _pooled(table, x)
    cor, inc = _heads(pooled, Wc, bc, W1, b1, W2, b2)
    return (cor, inc)
